# baseline (speedup 1.0000x reference)
import jax
import jax.numpy as jnp
from jax.experimental import pallas as pl
from jax.experimental.pallas import tpu as pltpu

N = 4096
IN_CH = 512
HID = 64
BLOCK = 512


def _body(seq_hbm, adj_ref, wt_ref, b_ref, a_ref, out_ref,
          fts_ref, seq_buf, seq_sem):
    i = pl.program_id(0)

    @pl.when(i == 0)
    def _():
        cp = pltpu.make_async_copy(seq_hbm, seq_buf, seq_sem)
        cp.start()
        cp.wait()
        fts = jnp.dot(
            seq_buf[...], wt_ref[...], preferred_element_type=jnp.float32
        )
        fts_ref[...] = fts.astype(jnp.bfloat16)

    out = jnp.dot(
        adj_ref[...].astype(jnp.bfloat16),
        fts_ref[...],
        preferred_element_type=jnp.float32,
    )
    out = out + b_ref[...]
    a = a_ref[0, 0]
    out_ref[...] = jnp.where(out > 0.0, out, a * out)


def kernel(seq, adj, W, bias, prelu_a):
    wt = W.T
    b2 = bias.reshape(1, HID)
    a2 = jnp.asarray(prelu_a, jnp.float32).reshape(1, 1)

    return pl.pallas_call(
        _body,
        grid=(N // BLOCK,),
        in_specs=[
            pl.BlockSpec(memory_space=pltpu.MemorySpace.HBM),
            pl.BlockSpec((BLOCK, N), lambda i: (i, 0)),
            pl.BlockSpec((IN_CH, HID), lambda i: (0, 0)),
            pl.BlockSpec((1, HID), lambda i: (0, 0)),
            pl.BlockSpec(memory_space=pltpu.SMEM),
        ],
        out_specs=pl.BlockSpec((BLOCK, HID), lambda i: (i, 0)),
        out_shape=jax.ShapeDtypeStruct((N, HID), jnp.float32),
        scratch_shapes=[
            pltpu.VMEM((N, HID), jnp.bfloat16),
            pltpu.VMEM((N, IN_CH), jnp.float32),
            pltpu.SemaphoreType.DMA,
        ],
    )(seq, adj, wt, b2, a2)


# speedup vs baseline: 1.0650x; 1.0650x over previous
import jax
import jax.numpy as jnp
from jax.experimental import pallas as pl
from jax.experimental.pallas import tpu as pltpu

N = 4096
IN_CH = 512
HID = 64
BLOCK = 1024


def _body(seq_ref, adj_ref, wt_ref, b_ref, a_ref, out_ref, fts_ref):
    i = pl.program_id(0)

    @pl.when(i == 0)
    def _():
        fts = jnp.dot(
            seq_ref[...], wt_ref[...], preferred_element_type=jnp.float32
        )
        fts_ref[...] = fts.astype(jnp.bfloat16)

    out = jnp.dot(
        adj_ref[...].astype(jnp.bfloat16),
        fts_ref[...],
        preferred_element_type=jnp.float32,
    )
    out = out + b_ref[...]
    a = a_ref[0, 0]
    out_ref[...] = jnp.where(out > 0.0, out, a * out)


def kernel(seq, adj, W, bias, prelu_a):
    wt = W.T
    b2 = bias.reshape(1, HID)
    a2 = jnp.asarray(prelu_a, jnp.float32).reshape(1, 1)

    grid = (N // BLOCK,)
    return pl.pallas_call(
        _body,
        grid=grid,
        in_specs=[
            pl.BlockSpec((N, IN_CH), lambda i: (0, 0)),
            pl.BlockSpec((BLOCK, N), lambda i: (i, 0)),
            pl.BlockSpec((IN_CH, HID), lambda i: (0, 0)),
            pl.BlockSpec((1, HID), lambda i: (0, 0)),
            pl.BlockSpec(memory_space=pltpu.SMEM),
        ],
        out_specs=pl.BlockSpec((BLOCK, HID), lambda i: (i, 0)),
        out_shape=jax.ShapeDtypeStruct((N, HID), jnp.float32),
        scratch_shapes=[pltpu.VMEM((N, HID), jnp.bfloat16)],
    )(seq, adj, wt, b2, a2)


# speedup vs baseline: 1.0704x; 1.0051x over previous
import jax
import jax.numpy as jnp
from jax.experimental import pallas as pl
from jax.experimental.pallas import tpu as pltpu

N = 4096
IN_CH = 512
HID = 64
BLOCK = 512


def _body(seq_ref, adj_ref, wt_ref, b_ref, a_ref, out_ref, fts_ref):
    i = pl.program_id(0)

    @pl.when(i == 0)
    def _():
        fts = jnp.dot(
            seq_ref[...], wt_ref[...], preferred_element_type=jnp.float32
        )
        fts_ref[...] = fts.astype(jnp.bfloat16)

    out = jnp.dot(
        adj_ref[...].astype(jnp.bfloat16),
        fts_ref[...],
        preferred_element_type=jnp.float32,
    )
    out = out + b_ref[...]
    a = a_ref[0, 0]
    out_ref[...] = jnp.where(out > 0.0, out, a * out)


def kernel(seq, adj, W, bias, prelu_a):
    wt = W.T
    b2 = bias.reshape(1, HID)
    a2 = jnp.asarray(prelu_a, jnp.float32).reshape(1, 1)

    grid = (N // BLOCK,)
    return pl.pallas_call(
        _body,
        grid=grid,
        in_specs=[
            pl.BlockSpec((N, IN_CH), lambda i: (0, 0)),
            pl.BlockSpec((BLOCK, N), lambda i: (i, 0)),
            pl.BlockSpec((IN_CH, HID), lambda i: (0, 0)),
            pl.BlockSpec((1, HID), lambda i: (0, 0)),
            pl.BlockSpec(memory_space=pltpu.SMEM),
        ],
        out_specs=pl.BlockSpec((BLOCK, HID), lambda i: (i, 0)),
        out_shape=jax.ShapeDtypeStruct((N, HID), jnp.float32),
        scratch_shapes=[pltpu.VMEM((N, HID), jnp.bfloat16)],
    )(seq, adj, wt, b2, a2)
